# baseline (speedup 1.0000x reference)
import jax
import jax.numpy as jnp
from jax import lax
from jax.experimental import pallas as pl
from jax.experimental.pallas import tpu as pltpu
from jax.experimental.pallas import tpu_sc as plsc

_f32 = jnp.float32
_i32 = jnp.int32

N = 10000
NP = 10240
E = 160000
KE = 128
NB = 80
EP = 16 * NB * KE
IN_CH, HID, OUT_CH = 256, 512, 256
RPT = NP // 16
ZR = 32
HB = NB // 2
BR = 1024
E32 = EP // 32
NV = E32 // 16

_mesh = plsc.VectorSubcoreMesh(core_axis_name="c", subcore_axis_name="s")


def _deg_body(dst_hbm, out_hbm, dstv, acc, tmp, total, shall):
    core = lax.axis_index("c")
    sub = lax.axis_index("s")
    wid = sub * 2 + core
    pltpu.sync_copy(dst_hbm.at[wid], dstv)
    zeros = jnp.zeros((16,), _f32)

    @pl.loop(0, NP // 16)
    def _(i):
        acc[pl.ds(i * 16, 16)] = zeros

    ones = jnp.ones((16,), _f32)

    @pl.loop(0, NV)
    def _(i):
        idx = dstv[pl.ds(i * 16, 16)]
        plsc.addupdate_scatter(acc, [idx], ones)

    pltpu.sync_copy(acc, shall.at[sub])
    plsc.subcore_barrier()

    @pl.loop(0, RPT // 16)
    def _(i):
        total[pl.ds(i * 16, 16)] = zeros

    for t in range(16):
        pltpu.sync_copy(shall.at[t].at[pl.ds(sub * RPT, RPT)], tmp)

        @pl.loop(0, RPT // 16)
        def _(i):
            sl = pl.ds(i * 16, 16)
            total[sl] = total[sl] + tmp[sl]

    pltpu.sync_copy(total, out_hbm.at[core].at[pl.ds(sub * RPT, RPT)])


_deg_kernel = pl.kernel(
    _deg_body,
    out_type=jax.ShapeDtypeStruct((2, NP), _f32),
    mesh=_mesh,
    compiler_params=pltpu.CompilerParams(needs_layout_passes=False),
    scratch_types=[
        pltpu.VMEM((E32,), _i32),
        pltpu.VMEM((NP,), _f32),
        pltpu.VMEM((RPT,), _f32),
        pltpu.VMEM((RPT,), _f32),
        pltpu.VMEM_SHARED((16, NP), _f32),
    ],
)


def _make_agg(n_chunks):
    cpc = n_chunks // 2

    def body(g_hbm, src_hbm, dst_hbm, out_hbm, srcv, dstv, gbuf0, gbuf1, zbuf,
             shacc, sem0, sem1):
        core = lax.axis_index("c")
        sub = lax.axis_index("s")
        my_src = src_hbm.at[sub]
        my_dst = dst_hbm.at[sub]
        zeros = jnp.zeros((16,), _f32)

        @pl.loop(0, ZR)
        def _(i):
            @pl.loop(0, 8)
            def _(k):
                zbuf[i, pl.ds(k * 16, 16)] = zeros

        for lc in range(cpc):
            c = core * cpc + lc
            gc = g_hbm.at[c]
            oc = out_hbm.at[c]

            pltpu.sync_copy(my_src.at[pl.ds(0, HB)], srcv)
            pltpu.sync_copy(my_dst.at[pl.ds(0, HB)], dstv)
            pltpu.async_copy(gc.at[srcv.at[0]], gbuf0, sem0)

            @pl.loop(0, RPT // ZR)
            def _(j):
                pltpu.sync_copy(zbuf, shacc.at[pl.ds(sub * RPT + j * ZR, ZR)])

            plsc.subcore_barrier()

            for h in range(2):
                if h:
                    pltpu.sync_copy(my_src.at[pl.ds(HB, HB)], srcv)
                    pltpu.sync_copy(my_dst.at[pl.ds(HB, HB)], dstv)
                    pltpu.async_copy(gc.at[srcv.at[0]], gbuf0, sem0)

                @pl.loop(0, HB)
                def _(b):
                    pltpu.make_async_copy(gc.at[srcv.at[b]], gbuf0, sem0).wait()
                    pltpu.sync_copy(gbuf0, shacc.at[dstv.at[b]], add=True)

                    @pl.when(b + 1 < HB)
                    def _():
                        pltpu.async_copy(gc.at[srcv.at[b + 1]], gbuf0, sem0)

            plsc.subcore_barrier()

            @pl.loop(0, RPT // ZR)
            def _(j):
                off = sub * RPT + j * ZR
                pltpu.sync_copy(shacc.at[pl.ds(off, ZR)], oc.at[pl.ds(off, ZR)])

    kern = pl.kernel(
        body,
        out_type=jax.ShapeDtypeStruct((n_chunks, NP, 128), _f32),
        mesh=_mesh,
        scratch_types=[
            pltpu.VMEM((HB, KE), _i32),
            pltpu.VMEM((HB, KE), _i32),
            pltpu.VMEM((KE, 128), _f32),
            pltpu.VMEM((KE, 128), _f32),
            pltpu.VMEM((ZR, 128), _f32),
            pltpu.VMEM_SHARED((NP, 128), _f32),
            pltpu.SemaphoreType.DMA,
            pltpu.SemaphoreType.DMA,
        ],
    )
    return kern


_agg4 = _make_agg(4)
_agg2 = _make_agg(2)


def _dis_block(degp_ref, r):
    rows = lax.broadcasted_iota(_i32, (BR, 1), 0) + r * BR
    deg = (degp_ref[0, :] + degp_ref[1, :]).reshape(BR, 1)
    return jnp.where(rows < N, lax.rsqrt(deg + 1.0), 0.0)


def _m1_body(x_ref, w_ref, degp_ref, o_ref):
    dis = _dis_block(degp_ref, pl.program_id(0))
    h = jnp.dot(x_ref[...], w_ref[...], preferred_element_type=_f32)
    o_ref[0] = h * dis


_m1 = pl.pallas_call(
    _m1_body,
    grid=(NP // BR, HID // 128),
    in_specs=[
        pl.BlockSpec((BR, IN_CH), lambda r, c: (r, 0)),
        pl.BlockSpec((IN_CH, 128), lambda r, c: (0, c)),
        pl.BlockSpec((2, BR), lambda r, c: (0, r)),
    ],
    out_specs=pl.BlockSpec((1, BR, 128), lambda r, c: (c, r, 0)),
    out_shape=jax.ShapeDtypeStruct((HID // 128, NP, 128), _f32),
)


def _m2_body(p_ref, g_ref, degp_ref, b_ref, w_ref, o_ref):
    dis = _dis_block(degp_ref, pl.program_id(0))
    acc = jnp.zeros((BR, 128), _f32)
    for cc in range(HID // 128):
        t = (p_ref[cc] + g_ref[cc]) * dis + b_ref[0, cc * 128:(cc + 1) * 128][None, :]
        t = jnp.maximum(t, 0.0)
        acc = acc + jnp.dot(t, w_ref[cc * 128:(cc + 1) * 128, :],
                            preferred_element_type=_f32)
    o_ref[0] = acc * dis


_m2 = pl.pallas_call(
    _m2_body,
    grid=(NP // BR, OUT_CH // 128),
    in_specs=[
        pl.BlockSpec((HID // 128, BR, 128), lambda r, c: (0, r, 0)),
        pl.BlockSpec((HID // 128, BR, 128), lambda r, c: (0, r, 0)),
        pl.BlockSpec((2, BR), lambda r, c: (0, r)),
        pl.BlockSpec((1, HID), lambda r, c: (0, 0)),
        pl.BlockSpec((HID, 128), lambda r, c: (0, c)),
    ],
    out_specs=pl.BlockSpec((1, BR, 128), lambda r, c: (c, r, 0)),
    out_shape=jax.ShapeDtypeStruct((OUT_CH // 128, NP, 128), _f32),
)


def _m3_body(p_ref, g_ref, degp_ref, b_ref, o_ref):
    dis = _dis_block(degp_ref, pl.program_id(0))
    t = (p_ref[0] + g_ref[0]) * dis + b_ref[0, :][None, :]
    o_ref[...] = jnp.maximum(t, 0.0)


_m3 = pl.pallas_call(
    _m3_body,
    grid=(NP // BR, OUT_CH // 128),
    in_specs=[
        pl.BlockSpec((1, BR, 128), lambda r, c: (c, r, 0)),
        pl.BlockSpec((1, BR, 128), lambda r, c: (c, r, 0)),
        pl.BlockSpec((2, BR), lambda r, c: (0, r)),
        pl.BlockSpec((1, 128), lambda r, c: (0, c)),
    ],
    out_specs=pl.BlockSpec((BR, 128), lambda r, c: (r, c)),
    out_shape=jax.ShapeDtypeStruct((NP, OUT_CH), _f32),
)


def kernel(x, edge_index, W1, b1, W2, b2):
    src = edge_index[0].astype(_i32)
    dst = edge_index[1].astype(_i32)
    pad = jnp.full((EP - E,), N, _i32)
    srcp = jnp.concatenate([src, pad]).reshape(16, NB, KE)
    dst_flat = jnp.concatenate([dst, pad])
    dstp = dst_flat.reshape(16, NB, KE)
    dst32 = dst_flat.reshape(32, E32)
    xp = jnp.concatenate([x.astype(_f32), jnp.zeros((NP - N, IN_CH), _f32)])

    degp = _deg_kernel(dst32)
    g1 = _m1(xp, W1, degp)
    p1 = _agg4(g1, srcp, dstp)
    g2 = _m2(p1, g1, degp, b1.reshape(1, HID), W2)
    p2 = _agg2(g2, srcp, dstp)
    outp = _m3(p2, g2, degp, b2.reshape(1, OUT_CH))
    return outp[:N]


# speedup vs baseline: 1.0020x; 1.0020x over previous
import jax
import jax.numpy as jnp
from jax import lax
from jax.experimental import pallas as pl
from jax.experimental.pallas import tpu as pltpu
from jax.experimental.pallas import tpu_sc as plsc

_f32 = jnp.float32
_i32 = jnp.int32

N = 10000
NP = 10240
E = 160000
KE = 128
NB = 80
EP = 16 * NB * KE
IN_CH, HID, OUT_CH = 256, 512, 256
RPT = NP // 16
ZR = 32
HB = NB // 2
BR = 1024
E32 = EP // 32
NV = E32 // 16

_mesh = plsc.VectorSubcoreMesh(core_axis_name="c", subcore_axis_name="s")


def _deg_body(dst_hbm, out_hbm, dstv, acc, tmp, total, shall):
    core = lax.axis_index("c")
    sub = lax.axis_index("s")
    wid = sub * 2 + core
    pltpu.sync_copy(dst_hbm.at[wid], dstv)
    zeros = jnp.zeros((16,), _f32)

    @pl.loop(0, NP // 16)
    def _(i):
        acc[pl.ds(i * 16, 16)] = zeros

    ones = jnp.ones((16,), _f32)

    @pl.loop(0, NV)
    def _(i):
        idx = dstv[pl.ds(i * 16, 16)]
        plsc.addupdate_scatter(acc, [idx], ones)

    pltpu.sync_copy(acc, shall.at[sub])
    plsc.subcore_barrier()

    @pl.loop(0, RPT // 16)
    def _(i):
        total[pl.ds(i * 16, 16)] = zeros

    for t in range(16):
        pltpu.sync_copy(shall.at[t].at[pl.ds(sub * RPT, RPT)], tmp)

        @pl.loop(0, RPT // 16)
        def _(i):
            sl = pl.ds(i * 16, 16)
            total[sl] = total[sl] + tmp[sl]

    pltpu.sync_copy(total, out_hbm.at[core].at[pl.ds(sub * RPT, RPT)])


_deg_kernel = pl.kernel(
    _deg_body,
    out_type=jax.ShapeDtypeStruct((2, NP), _f32),
    mesh=_mesh,
    compiler_params=pltpu.CompilerParams(needs_layout_passes=False),
    scratch_types=[
        pltpu.VMEM((E32,), _i32),
        pltpu.VMEM((NP,), _f32),
        pltpu.VMEM((RPT,), _f32),
        pltpu.VMEM((RPT,), _f32),
        pltpu.VMEM_SHARED((16, NP), _f32),
    ],
)


def _make_agg(n_chunks):
    cpc = n_chunks // 2

    def body(g_hbm, src_hbm, dst_hbm, out_hbm, srcv, dstv, gbuf, zbuf,
             shacc, sem):
        core = lax.axis_index("c")
        sub = lax.axis_index("s")
        pltpu.sync_copy(src_hbm.at[sub], srcv)
        pltpu.sync_copy(dst_hbm.at[sub], dstv)
        zeros = jnp.zeros((16,), _f32)

        @pl.loop(0, ZR)
        def _(i):
            @pl.loop(0, 8)
            def _(k):
                zbuf[i, pl.ds(k * 16, 16)] = zeros

        for lc in range(cpc):
            c = core * cpc + lc
            gc = g_hbm.at[c]
            oc = out_hbm.at[c]

            @pl.loop(0, RPT // ZR)
            def _(j):
                pltpu.sync_copy(zbuf, shacc.at[pl.ds(sub * RPT + j * ZR, ZR)])

            plsc.subcore_barrier()

            @pl.loop(0, NB)
            def _(b):
                pltpu.async_copy(gc.at[srcv.at[b]], gbuf, sem).wait()
                pltpu.sync_copy(gbuf, shacc.at[dstv.at[b]], add=True)

            plsc.subcore_barrier()

            @pl.loop(0, RPT // ZR)
            def _(j):
                off = sub * RPT + j * ZR
                pltpu.sync_copy(shacc.at[pl.ds(off, ZR)], oc.at[pl.ds(off, ZR)])

    kern = pl.kernel(
        body,
        out_type=jax.ShapeDtypeStruct((n_chunks, NP, 128), _f32),
        mesh=_mesh,
        scratch_types=[
            pltpu.VMEM((NB, KE), _i32),
            pltpu.VMEM((NB, KE), _i32),
            pltpu.VMEM((KE, 128), _f32),
            pltpu.VMEM((ZR, 128), _f32),
            pltpu.VMEM_SHARED((NP, 128), _f32),
            pltpu.SemaphoreType.DMA,
        ],
    )
    return kern


_agg4 = _make_agg(4)
_agg2 = _make_agg(2)


def _dis_block(degp_ref, r):
    rows = lax.broadcasted_iota(_i32, (BR, 1), 0) + r * BR
    deg = (degp_ref[0, :] + degp_ref[1, :]).reshape(BR, 1)
    return jnp.where(rows < N, lax.rsqrt(deg + 1.0), 0.0)


def _m1_body(x_ref, w_ref, degp_ref, o_ref):
    dis = _dis_block(degp_ref, pl.program_id(0))
    h = jnp.dot(x_ref[...], w_ref[...], preferred_element_type=_f32)
    o_ref[0] = h * dis


_m1 = pl.pallas_call(
    _m1_body,
    grid=(NP // BR, HID // 128),
    in_specs=[
        pl.BlockSpec((BR, IN_CH), lambda r, c: (r, 0)),
        pl.BlockSpec((IN_CH, 128), lambda r, c: (0, c)),
        pl.BlockSpec((2, BR), lambda r, c: (0, r)),
    ],
    out_specs=pl.BlockSpec((1, BR, 128), lambda r, c: (c, r, 0)),
    out_shape=jax.ShapeDtypeStruct((HID // 128, NP, 128), _f32),
)


def _m2_body(p_ref, g_ref, degp_ref, b_ref, w_ref, o_ref):
    dis = _dis_block(degp_ref, pl.program_id(0))
    acc = jnp.zeros((BR, 128), _f32)
    for cc in range(HID // 128):
        t = (p_ref[cc] + g_ref[cc]) * dis + b_ref[0, cc * 128:(cc + 1) * 128][None, :]
        t = jnp.maximum(t, 0.0)
        acc = acc + jnp.dot(t, w_ref[cc * 128:(cc + 1) * 128, :],
                            preferred_element_type=_f32)
    o_ref[0] = acc * dis


_m2 = pl.pallas_call(
    _m2_body,
    grid=(NP // BR, OUT_CH // 128),
    in_specs=[
        pl.BlockSpec((HID // 128, BR, 128), lambda r, c: (0, r, 0)),
        pl.BlockSpec((HID // 128, BR, 128), lambda r, c: (0, r, 0)),
        pl.BlockSpec((2, BR), lambda r, c: (0, r)),
        pl.BlockSpec((1, HID), lambda r, c: (0, 0)),
        pl.BlockSpec((HID, 128), lambda r, c: (0, c)),
    ],
    out_specs=pl.BlockSpec((1, BR, 128), lambda r, c: (c, r, 0)),
    out_shape=jax.ShapeDtypeStruct((OUT_CH // 128, NP, 128), _f32),
)


def _m3_body(p_ref, g_ref, degp_ref, b_ref, o_ref):
    dis = _dis_block(degp_ref, pl.program_id(0))
    t = (p_ref[0] + g_ref[0]) * dis + b_ref[0, :][None, :]
    o_ref[...] = jnp.maximum(t, 0.0)


_m3 = pl.pallas_call(
    _m3_body,
    grid=(NP // BR, OUT_CH // 128),
    in_specs=[
        pl.BlockSpec((1, BR, 128), lambda r, c: (c, r, 0)),
        pl.BlockSpec((1, BR, 128), lambda r, c: (c, r, 0)),
        pl.BlockSpec((2, BR), lambda r, c: (0, r)),
        pl.BlockSpec((1, 128), lambda r, c: (0, c)),
    ],
    out_specs=pl.BlockSpec((BR, 128), lambda r, c: (r, c)),
    out_shape=jax.ShapeDtypeStruct((NP, OUT_CH), _f32),
)


def kernel(x, edge_index, W1, b1, W2, b2):
    src = edge_index[0].astype(_i32)
    dst = edge_index[1].astype(_i32)
    pad = jnp.full((EP - E,), N, _i32)
    srcp = jnp.concatenate([src, pad]).reshape(16, NB, KE)
    dst_flat = jnp.concatenate([dst, pad])
    dstp = dst_flat.reshape(16, NB, KE)
    dst32 = dst_flat.reshape(32, E32)
    xp = jnp.concatenate([x.astype(_f32), jnp.zeros((NP - N, IN_CH), _f32)])

    degp = _deg_kernel(dst32)
    g1 = _m1(xp, W1, degp)
    p1 = _agg4(g1, srcp, dstp)
    g2 = _m2(p1, g1, degp, b1.reshape(1, HID), W2)
    p2 = _agg2(g2, srcp, dstp)
    outp = _m3(p2, g2, degp, b2.reshape(1, OUT_CH))
    return outp[:N]


# speedup vs baseline: 1.0161x; 1.0140x over previous
import jax
import jax.numpy as jnp
from jax import lax
from jax.experimental import pallas as pl
from jax.experimental.pallas import tpu as pltpu
from jax.experimental.pallas import tpu_sc as plsc

_f32 = jnp.float32
_i32 = jnp.int32

N = 10000
NP = 10240
E = 160000
KE = 128
NB = 80
EP = 16 * NB * KE
IN_CH, HID, OUT_CH = 256, 512, 256
RPT = NP // 16
ZR = 64
HB = NB // 2
BR = 1024
E32 = EP // 32
NV = E32 // 16

_mesh = plsc.VectorSubcoreMesh(core_axis_name="c", subcore_axis_name="s")


def _deg_body(dst_hbm, out_hbm, dstv, acc, tmp, total, shall):
    core = lax.axis_index("c")
    sub = lax.axis_index("s")
    wid = sub * 2 + core
    pltpu.sync_copy(dst_hbm.at[wid], dstv)
    zeros = jnp.zeros((16,), _f32)

    @pl.loop(0, NP // 16)
    def _(i):
        acc[pl.ds(i * 16, 16)] = zeros

    ones = jnp.ones((16,), _f32)

    @pl.loop(0, NV)
    def _(i):
        idx = dstv[pl.ds(i * 16, 16)]
        plsc.addupdate_scatter(acc, [idx], ones)

    pltpu.sync_copy(acc, shall.at[sub])
    plsc.subcore_barrier()

    @pl.loop(0, RPT // 16)
    def _(i):
        total[pl.ds(i * 16, 16)] = zeros

    for t in range(16):
        pltpu.sync_copy(shall.at[t].at[pl.ds(sub * RPT, RPT)], tmp)

        @pl.loop(0, RPT // 16)
        def _(i):
            sl = pl.ds(i * 16, 16)
            total[sl] = total[sl] + tmp[sl]

    pltpu.sync_copy(total, out_hbm.at[core].at[pl.ds(sub * RPT, RPT)])


_deg_kernel = pl.kernel(
    _deg_body,
    out_type=jax.ShapeDtypeStruct((2, NP), _f32),
    mesh=_mesh,
    compiler_params=pltpu.CompilerParams(needs_layout_passes=False),
    scratch_types=[
        pltpu.VMEM((E32,), _i32),
        pltpu.VMEM((NP,), _f32),
        pltpu.VMEM((RPT,), _f32),
        pltpu.VMEM((RPT,), _f32),
        pltpu.VMEM_SHARED((16, NP), _f32),
    ],
)


def _make_agg(n_chunks):
    cpc = n_chunks // 2

    def body(g_hbm, src_hbm, dst_hbm, out_hbm, srcv, dstv, gbuf, zbuf,
             shacc, sem):
        core = lax.axis_index("c")
        sub = lax.axis_index("s")
        pltpu.sync_copy(src_hbm.at[sub], srcv)
        pltpu.sync_copy(dst_hbm.at[sub], dstv)
        zeros = jnp.zeros((16,), _f32)

        @pl.loop(0, ZR)
        def _(i):
            @pl.loop(0, 8)
            def _(k):
                zbuf[i, pl.ds(k * 16, 16)] = zeros

        for lc in range(cpc):
            c = core * cpc + lc
            gc = g_hbm.at[c]
            oc = out_hbm.at[c]

            @pl.loop(0, RPT // ZR)
            def _(j):
                pltpu.sync_copy(zbuf, shacc.at[pl.ds(sub * RPT + j * ZR, ZR)])

            plsc.subcore_barrier()

            @pl.loop(0, NB)
            def _(b):
                pltpu.async_copy(gc.at[srcv.at[b]], gbuf, sem).wait()
                pltpu.sync_copy(gbuf, shacc.at[dstv.at[b]], add=True)

            plsc.subcore_barrier()

            @pl.loop(0, RPT // ZR)
            def _(j):
                off = sub * RPT + j * ZR
                pltpu.sync_copy(shacc.at[pl.ds(off, ZR)], oc.at[pl.ds(off, ZR)])

    kern = pl.kernel(
        body,
        out_type=jax.ShapeDtypeStruct((n_chunks, NP, 128), _f32),
        mesh=_mesh,
        scratch_types=[
            pltpu.VMEM((NB, KE), _i32),
            pltpu.VMEM((NB, KE), _i32),
            pltpu.VMEM((KE, 128), _f32),
            pltpu.VMEM((ZR, 128), _f32),
            pltpu.VMEM_SHARED((NP, 128), _f32),
            pltpu.SemaphoreType.DMA,
        ],
    )
    return kern


_agg4 = _make_agg(4)
_agg2 = _make_agg(2)


def _dis_block(degp_ref, r):
    rows = lax.broadcasted_iota(_i32, (BR, 1), 0) + r * BR
    deg = (degp_ref[0, :] + degp_ref[1, :]).reshape(BR, 1)
    return jnp.where(rows < N, lax.rsqrt(deg + 1.0), 0.0)


def _m1_body(x_ref, w_ref, degp_ref, o_ref):
    dis = _dis_block(degp_ref, pl.program_id(0))
    h = jnp.dot(x_ref[...], w_ref[...], preferred_element_type=_f32)
    o_ref[0] = h * dis


_m1 = pl.pallas_call(
    _m1_body,
    grid=(NP // BR, HID // 128),
    in_specs=[
        pl.BlockSpec((BR, IN_CH), lambda r, c: (r, 0)),
        pl.BlockSpec((IN_CH, 128), lambda r, c: (0, c)),
        pl.BlockSpec((2, BR), lambda r, c: (0, r)),
    ],
    out_specs=pl.BlockSpec((1, BR, 128), lambda r, c: (c, r, 0)),
    out_shape=jax.ShapeDtypeStruct((HID // 128, NP, 128), _f32),
)


def _m2_body(p_ref, g_ref, degp_ref, b_ref, w_ref, o_ref):
    dis = _dis_block(degp_ref, pl.program_id(0))
    acc = jnp.zeros((BR, 128), _f32)
    for cc in range(HID // 128):
        t = (p_ref[cc] + g_ref[cc]) * dis + b_ref[0, cc * 128:(cc + 1) * 128][None, :]
        t = jnp.maximum(t, 0.0)
        acc = acc + jnp.dot(t, w_ref[cc * 128:(cc + 1) * 128, :],
                            preferred_element_type=_f32)
    o_ref[0] = acc * dis


_m2 = pl.pallas_call(
    _m2_body,
    grid=(NP // BR, OUT_CH // 128),
    in_specs=[
        pl.BlockSpec((HID // 128, BR, 128), lambda r, c: (0, r, 0)),
        pl.BlockSpec((HID // 128, BR, 128), lambda r, c: (0, r, 0)),
        pl.BlockSpec((2, BR), lambda r, c: (0, r)),
        pl.BlockSpec((1, HID), lambda r, c: (0, 0)),
        pl.BlockSpec((HID, 128), lambda r, c: (0, c)),
    ],
    out_specs=pl.BlockSpec((1, BR, 128), lambda r, c: (c, r, 0)),
    out_shape=jax.ShapeDtypeStruct((OUT_CH // 128, NP, 128), _f32),
)


def _m3_body(p_ref, g_ref, degp_ref, b_ref, o_ref):
    dis = _dis_block(degp_ref, pl.program_id(0))
    t = (p_ref[0] + g_ref[0]) * dis + b_ref[0, :][None, :]
    o_ref[...] = jnp.maximum(t, 0.0)


_m3 = pl.pallas_call(
    _m3_body,
    grid=(NP // BR, OUT_CH // 128),
    in_specs=[
        pl.BlockSpec((1, BR, 128), lambda r, c: (c, r, 0)),
        pl.BlockSpec((1, BR, 128), lambda r, c: (c, r, 0)),
        pl.BlockSpec((2, BR), lambda r, c: (0, r)),
        pl.BlockSpec((1, 128), lambda r, c: (0, c)),
    ],
    out_specs=pl.BlockSpec((BR, 128), lambda r, c: (r, c)),
    out_shape=jax.ShapeDtypeStruct((NP, OUT_CH), _f32),
)


def kernel(x, edge_index, W1, b1, W2, b2):
    src = edge_index[0].astype(_i32)
    dst = edge_index[1].astype(_i32)
    pad = jnp.full((EP - E,), N, _i32)
    srcp = jnp.concatenate([src, pad]).reshape(16, NB, KE)
    dst_flat = jnp.concatenate([dst, pad])
    dstp = dst_flat.reshape(16, NB, KE)
    dst32 = dst_flat.reshape(32, E32)
    xp = jnp.concatenate([x.astype(_f32), jnp.zeros((NP - N, IN_CH), _f32)])

    degp = _deg_kernel(dst32)
    g1 = _m1(xp, W1, degp)
    p1 = _agg4(g1, srcp, dstp)
    g2 = _m2(p1, g1, degp, b1.reshape(1, HID), W2)
    p2 = _agg2(g2, srcp, dstp)
    outp = _m3(p2, g2, degp, b2.reshape(1, OUT_CH))
    return outp[:N]


# speedup vs baseline: 1.7039x; 1.6769x over previous
import jax
import jax.numpy as jnp
from jax import lax
from jax.experimental import pallas as pl
from jax.experimental.pallas import tpu as pltpu
from jax.experimental.pallas import tpu_sc as plsc

_f32 = jnp.float32
_i32 = jnp.int32

N = 10000
NP = 10240
E = 160000
KE = 128
NB = 80
EP = 16 * NB * KE
IN_CH, HID, OUT_CH = 256, 512, 256
RPT = NP // 16
ZR = 64
HB = NB // 2
BR = 1024
E32 = EP // 32
NV = E32 // 16

_mesh = plsc.VectorSubcoreMesh(core_axis_name="c", subcore_axis_name="s")


def _deg_body(dst_hbm, out_hbm, dstv, acc, tmp, total, shall):
    core = lax.axis_index("c")
    sub = lax.axis_index("s")
    wid = sub * 2 + core
    pltpu.sync_copy(dst_hbm.at[wid], dstv)
    zeros = jnp.zeros((16,), _f32)

    @pl.loop(0, NP // 16)
    def _(i):
        acc[pl.ds(i * 16, 16)] = zeros

    ones = jnp.ones((16,), _f32)

    @pl.loop(0, NV)
    def _(i):
        idx = dstv[pl.ds(i * 16, 16)]
        plsc.addupdate_scatter(acc, [idx], ones)

    pltpu.sync_copy(acc, shall.at[sub])
    plsc.subcore_barrier()

    @pl.loop(0, RPT // 16)
    def _(i):
        total[pl.ds(i * 16, 16)] = zeros

    for t in range(16):
        pltpu.sync_copy(shall.at[t].at[pl.ds(sub * RPT, RPT)], tmp)

        @pl.loop(0, RPT // 16)
        def _(i):
            sl = pl.ds(i * 16, 16)
            total[sl] = total[sl] + tmp[sl]

    pltpu.sync_copy(total, out_hbm.at[core].at[pl.ds(sub * RPT, RPT)])


_deg_kernel = pl.kernel(
    _deg_body,
    out_type=jax.ShapeDtypeStruct((2, NP), _f32),
    mesh=_mesh,
    compiler_params=pltpu.CompilerParams(needs_layout_passes=False),
    scratch_types=[
        pltpu.VMEM((E32,), _i32),
        pltpu.VMEM((NP,), _f32),
        pltpu.VMEM((RPT,), _f32),
        pltpu.VMEM((RPT,), _f32),
        pltpu.VMEM_SHARED((16, NP), _f32),
    ],
)


def _make_agg(n_chunks):
    cpc = n_chunks // 2

    def body(g_hbm, src_hbm, dst_hbm, out_hbm, srcv, dstv, gbuf, zbuf,
             shacc, sem):
        core = lax.axis_index("c")
        sub = lax.axis_index("s")
        pltpu.sync_copy(src_hbm.at[sub], srcv)
        pltpu.sync_copy(dst_hbm.at[sub], dstv)
        zeros = jnp.zeros((16,), _f32)

        @pl.loop(0, ZR)
        def _(i):
            @pl.loop(0, 8)
            def _(k):
                zbuf[i, pl.ds(k * 16, 16)] = zeros

        for lc in range(cpc):
            c = core * cpc + lc
            gc = g_hbm.at[c]
            oc = out_hbm.at[c]

            @pl.loop(0, RPT // ZR)
            def _(j):
                pltpu.sync_copy(zbuf, shacc.at[pl.ds(sub * RPT + j * ZR, ZR)])

            plsc.subcore_barrier()

            @pl.loop(0, NB)
            def _(b):
                pltpu.async_copy(gc.at[srcv.at[b]], gbuf, sem).wait()
                pltpu.sync_copy(gbuf, shacc.at[dstv.at[b]], add=True)

            plsc.subcore_barrier()

            @pl.loop(0, RPT // ZR)
            def _(j):
                off = sub * RPT + j * ZR
                pltpu.sync_copy(shacc.at[pl.ds(off, ZR)], oc.at[pl.ds(off, ZR)])

    kern = pl.kernel(
        body,
        out_type=jax.ShapeDtypeStruct((n_chunks, NP, 128), _f32),
        mesh=_mesh,
        scratch_types=[
            pltpu.VMEM((NB, KE), _i32),
            pltpu.VMEM((NB, KE), _i32),
            pltpu.VMEM((KE, 128), _f32),
            pltpu.VMEM((ZR, 128), _f32),
            pltpu.VMEM_SHARED((NP, 128), _f32),
            pltpu.SemaphoreType.DMA,
        ],
    )
    return kern


_agg4 = _make_agg(4)
_agg2 = _make_agg(2)


def _dis_block(degp_ref, r):
    rows = lax.broadcasted_iota(_i32, (BR, 1), 0) + r * BR
    deg = (degp_ref[0, :] + degp_ref[1, :]).reshape(BR, 1)
    return jnp.where(rows < N, lax.rsqrt(deg + 1.0), 0.0)


def _m1_body(x_ref, w_ref, degp_ref, o_ref):
    dis = _dis_block(degp_ref, pl.program_id(0))
    h = jnp.dot(x_ref[...], w_ref[...], preferred_element_type=_f32)
    o_ref[0] = h * dis


_m1 = pl.pallas_call(
    _m1_body,
    grid=(NP // BR, HID // 128),
    in_specs=[
        pl.BlockSpec((BR, IN_CH), lambda r, c: (r, 0)),
        pl.BlockSpec((IN_CH, 128), lambda r, c: (0, c)),
        pl.BlockSpec((2, BR), lambda r, c: (0, r)),
    ],
    out_specs=pl.BlockSpec((1, BR, 128), lambda r, c: (c, r, 0)),
    out_shape=jax.ShapeDtypeStruct((HID // 128, NP, 128), _f32),
)


def _m2_body(p_ref, g_ref, degp_ref, b_ref, w_ref, o_ref):
    dis = _dis_block(degp_ref, pl.program_id(0))
    acc = jnp.zeros((BR, 128), _f32)
    for cc in range(HID // 128):
        t = (p_ref[cc] + g_ref[cc]) * dis + b_ref[0, cc * 128:(cc + 1) * 128][None, :]
        t = jnp.maximum(t, 0.0)
        acc = acc + jnp.dot(t, w_ref[cc * 128:(cc + 1) * 128, :],
                            preferred_element_type=_f32)
    o_ref[0] = acc * dis


_m2 = pl.pallas_call(
    _m2_body,
    grid=(NP // BR, OUT_CH // 128),
    in_specs=[
        pl.BlockSpec((HID // 128, BR, 128), lambda r, c: (0, r, 0)),
        pl.BlockSpec((HID // 128, BR, 128), lambda r, c: (0, r, 0)),
        pl.BlockSpec((2, BR), lambda r, c: (0, r)),
        pl.BlockSpec((1, HID), lambda r, c: (0, 0)),
        pl.BlockSpec((HID, 128), lambda r, c: (0, c)),
    ],
    out_specs=pl.BlockSpec((1, BR, 128), lambda r, c: (c, r, 0)),
    out_shape=jax.ShapeDtypeStruct((OUT_CH // 128, NP, 128), _f32),
)


def _m3_body(p_ref, g_ref, degp_ref, b_ref, o_ref):
    dis = _dis_block(degp_ref, pl.program_id(0))
    t = (p_ref[0] + g_ref[0]) * dis + b_ref[0, :][None, :]
    o_ref[...] = jnp.maximum(t, 0.0)


_m3 = pl.pallas_call(
    _m3_body,
    grid=(NP // BR, OUT_CH // 128),
    in_specs=[
        pl.BlockSpec((1, BR, 128), lambda r, c: (c, r, 0)),
        pl.BlockSpec((1, BR, 128), lambda r, c: (c, r, 0)),
        pl.BlockSpec((2, BR), lambda r, c: (0, r)),
        pl.BlockSpec((1, 128), lambda r, c: (0, c)),
    ],
    out_specs=pl.BlockSpec((BR, 128), lambda r, c: (r, c)),
    out_shape=jax.ShapeDtypeStruct((NP, OUT_CH), _f32),
)


def kernel(x, edge_index, W1, b1, W2, b2):
    src = edge_index[0].astype(_i32)
    dst = edge_index[1].astype(_i32)
    pad = N + (jnp.arange(EP - E, dtype=_i32) % (NP - N))
    srcp = jnp.concatenate([src, pad]).reshape(16, NB, KE)
    dst_flat = jnp.concatenate([dst, pad])
    dstp = dst_flat.reshape(16, NB, KE)
    dst32 = dst_flat.reshape(32, E32)
    xp = jnp.concatenate([x.astype(_f32), jnp.zeros((NP - N, IN_CH), _f32)])

    degp = _deg_kernel(dst32)
    g1 = _m1(xp, W1, degp)
    p1 = _agg4(g1, srcp, dstp)
    g2 = _m2(p1, g1, degp, b1.reshape(1, HID), W2)
    p2 = _agg2(g2, srcp, dstp)
    outp = _m3(p2, g2, degp, b2.reshape(1, OUT_CH))
    return outp[:N]


# speedup vs baseline: 2.3663x; 1.3888x over previous
import jax
import jax.numpy as jnp
from jax import lax
from jax.experimental import pallas as pl
from jax.experimental.pallas import tpu as pltpu
from jax.experimental.pallas import tpu_sc as plsc

_f32 = jnp.float32
_i32 = jnp.int32

N = 10000
NP = 10240
E = 160000
KE = 128
NB = 80
EP = 16 * NB * KE
IN_CH, HID, OUT_CH = 256, 512, 256
RPT = NP // 16
ZR = 32
HB = NB // 2
BR = 1024
E32 = EP // 32
NV = E32 // 16

_mesh = plsc.VectorSubcoreMesh(core_axis_name="c", subcore_axis_name="s")


def _deg_body(dst_hbm, out_hbm, dstv, acc, tmp, total, shall):
    core = lax.axis_index("c")
    sub = lax.axis_index("s")
    wid = sub * 2 + core
    pltpu.sync_copy(dst_hbm.at[wid], dstv)
    zeros = jnp.zeros((16,), _f32)

    @pl.loop(0, NP // 16)
    def _(i):
        acc[pl.ds(i * 16, 16)] = zeros

    ones = jnp.ones((16,), _f32)

    @pl.loop(0, NV)
    def _(i):
        idx = dstv[pl.ds(i * 16, 16)]
        plsc.addupdate_scatter(acc, [idx], ones)

    pltpu.sync_copy(acc, shall.at[sub])
    plsc.subcore_barrier()

    @pl.loop(0, RPT // 16)
    def _(i):
        total[pl.ds(i * 16, 16)] = zeros

    for t in range(16):
        pltpu.sync_copy(shall.at[t].at[pl.ds(sub * RPT, RPT)], tmp)

        @pl.loop(0, RPT // 16)
        def _(i):
            sl = pl.ds(i * 16, 16)
            total[sl] = total[sl] + tmp[sl]

    pltpu.sync_copy(total, out_hbm.at[core].at[pl.ds(sub * RPT, RPT)])


_deg_kernel = pl.kernel(
    _deg_body,
    out_type=jax.ShapeDtypeStruct((2, NP), _f32),
    mesh=_mesh,
    compiler_params=pltpu.CompilerParams(needs_layout_passes=False),
    scratch_types=[
        pltpu.VMEM((E32,), _i32),
        pltpu.VMEM((NP,), _f32),
        pltpu.VMEM((RPT,), _f32),
        pltpu.VMEM((RPT,), _f32),
        pltpu.VMEM_SHARED((16, NP), _f32),
    ],
)


def _make_agg(n_chunks):
    cpc = n_chunks // 2

    def body(g_hbm, src_hbm, dst_hbm, out_hbm, srcv, dstv, gbuf0, gbuf1, zbuf,
             shacc, sem0, sem1):
        core = lax.axis_index("c")
        sub = lax.axis_index("s")
        my_src = src_hbm.at[sub]
        my_dst = dst_hbm.at[sub]
        zeros = jnp.zeros((16,), _f32)

        @pl.loop(0, ZR)
        def _(i):
            @pl.loop(0, 8)
            def _(k):
                zbuf[i, pl.ds(k * 16, 16)] = zeros

        def gwait(b, buf, sem):
            pltpu.make_async_copy(gc.at[srcv.at[b]], buf, sem).wait()

        def scat(b, buf):
            pltpu.sync_copy(buf, shacc.at[dstv.at[b]], add=True)

        for lc in range(cpc):
            c = core * cpc + lc
            gc = g_hbm.at[c]
            oc = out_hbm.at[c]

            pltpu.sync_copy(my_src.at[pl.ds(0, HB)], srcv)
            pltpu.sync_copy(my_dst.at[pl.ds(0, HB)], dstv)
            pltpu.async_copy(gc.at[srcv.at[0]], gbuf0, sem0)

            @pl.loop(0, RPT // ZR)
            def _(j):
                pltpu.sync_copy(zbuf, shacc.at[pl.ds(sub * RPT + j * ZR, ZR)])

            plsc.subcore_barrier()

            for h in range(2):
                if h:
                    pltpu.sync_copy(my_src.at[pl.ds(HB, HB)], srcv)
                    pltpu.sync_copy(my_dst.at[pl.ds(HB, HB)], dstv)
                    pltpu.async_copy(gc.at[srcv.at[0]], gbuf0, sem0)

                @pl.loop(0, HB // 2 - 1)
                def _(i):
                    b = i * 2
                    pltpu.async_copy(gc.at[srcv.at[b + 1]], gbuf1, sem1)
                    gwait(b, gbuf0, sem0)
                    scat(b, gbuf0)
                    pltpu.async_copy(gc.at[srcv.at[b + 2]], gbuf0, sem0)
                    gwait(b + 1, gbuf1, sem1)
                    scat(b + 1, gbuf1)

                pltpu.async_copy(gc.at[srcv.at[HB - 1]], gbuf1, sem1)
                gwait(HB - 2, gbuf0, sem0)
                scat(HB - 2, gbuf0)
                gwait(HB - 1, gbuf1, sem1)
                scat(HB - 1, gbuf1)

            plsc.subcore_barrier()

            @pl.loop(0, RPT // ZR)
            def _(j):
                off = sub * RPT + j * ZR
                pltpu.sync_copy(shacc.at[pl.ds(off, ZR)], oc.at[pl.ds(off, ZR)])

    kern = pl.kernel(
        body,
        out_type=jax.ShapeDtypeStruct((n_chunks, NP, 128), _f32),
        mesh=_mesh,
        scratch_types=[
            pltpu.VMEM((HB, KE), _i32),
            pltpu.VMEM((HB, KE), _i32),
            pltpu.VMEM((KE, 128), _f32),
            pltpu.VMEM((KE, 128), _f32),
            pltpu.VMEM((ZR, 128), _f32),
            pltpu.VMEM_SHARED((NP, 128), _f32),
            pltpu.SemaphoreType.DMA,
            pltpu.SemaphoreType.DMA,
        ],
    )
    return kern


_agg4 = _make_agg(4)
_agg2 = _make_agg(2)


def _dis_block(degp_ref, r):
    rows = lax.broadcasted_iota(_i32, (BR, 1), 0) + r * BR
    deg = (degp_ref[0, :] + degp_ref[1, :]).reshape(BR, 1)
    return jnp.where(rows < N, lax.rsqrt(deg + 1.0), 0.0)


def _m1_body(x_ref, w_ref, degp_ref, o_ref):
    dis = _dis_block(degp_ref, pl.program_id(0))
    h = jnp.dot(x_ref[...], w_ref[...], preferred_element_type=_f32)
    o_ref[0] = h * dis


_m1 = pl.pallas_call(
    _m1_body,
    grid=(NP // BR, HID // 128),
    in_specs=[
        pl.BlockSpec((BR, IN_CH), lambda r, c: (r, 0)),
        pl.BlockSpec((IN_CH, 128), lambda r, c: (0, c)),
        pl.BlockSpec((2, BR), lambda r, c: (0, r)),
    ],
    out_specs=pl.BlockSpec((1, BR, 128), lambda r, c: (c, r, 0)),
    out_shape=jax.ShapeDtypeStruct((HID // 128, NP, 128), _f32),
)


def _m2_body(p_ref, g_ref, degp_ref, b_ref, w_ref, o_ref):
    dis = _dis_block(degp_ref, pl.program_id(0))
    acc = jnp.zeros((BR, 128), _f32)
    for cc in range(HID // 128):
        t = (p_ref[cc] + g_ref[cc]) * dis + b_ref[0, cc * 128:(cc + 1) * 128][None, :]
        t = jnp.maximum(t, 0.0)
        acc = acc + jnp.dot(t, w_ref[cc * 128:(cc + 1) * 128, :],
                            preferred_element_type=_f32)
    o_ref[0] = acc * dis


_m2 = pl.pallas_call(
    _m2_body,
    grid=(NP // BR, OUT_CH // 128),
    in_specs=[
        pl.BlockSpec((HID // 128, BR, 128), lambda r, c: (0, r, 0)),
        pl.BlockSpec((HID // 128, BR, 128), lambda r, c: (0, r, 0)),
        pl.BlockSpec((2, BR), lambda r, c: (0, r)),
        pl.BlockSpec((1, HID), lambda r, c: (0, 0)),
        pl.BlockSpec((HID, 128), lambda r, c: (0, c)),
    ],
    out_specs=pl.BlockSpec((1, BR, 128), lambda r, c: (c, r, 0)),
    out_shape=jax.ShapeDtypeStruct((OUT_CH // 128, NP, 128), _f32),
)


def _m3_body(p_ref, g_ref, degp_ref, b_ref, o_ref):
    dis = _dis_block(degp_ref, pl.program_id(0))
    t = (p_ref[0] + g_ref[0]) * dis + b_ref[0, :][None, :]
    o_ref[...] = jnp.maximum(t, 0.0)


_m3 = pl.pallas_call(
    _m3_body,
    grid=(NP // BR, OUT_CH // 128),
    in_specs=[
        pl.BlockSpec((1, BR, 128), lambda r, c: (c, r, 0)),
        pl.BlockSpec((1, BR, 128), lambda r, c: (c, r, 0)),
        pl.BlockSpec((2, BR), lambda r, c: (0, r)),
        pl.BlockSpec((1, 128), lambda r, c: (0, c)),
    ],
    out_specs=pl.BlockSpec((BR, 128), lambda r, c: (r, c)),
    out_shape=jax.ShapeDtypeStruct((NP, OUT_CH), _f32),
)


def kernel(x, edge_index, W1, b1, W2, b2):
    src = edge_index[0].astype(_i32)
    dst = edge_index[1].astype(_i32)
    pad = N + (jnp.arange(EP - E, dtype=_i32) % (NP - N))
    srcp = jnp.concatenate([src, pad]).reshape(16, NB, KE)
    dst_flat = jnp.concatenate([dst, pad])
    dstp = dst_flat.reshape(16, NB, KE)
    dst32 = dst_flat.reshape(32, E32)
    xp = jnp.concatenate([x.astype(_f32), jnp.zeros((NP - N, IN_CH), _f32)])

    degp = _deg_kernel(dst32)
    g1 = _m1(xp, W1, degp)
    p1 = _agg4(g1, srcp, dstp)
    g2 = _m2(p1, g1, degp, b1.reshape(1, HID), W2)
    p2 = _agg2(g2, srcp, dstp)
    outp = _m3(p2, g2, degp, b2.reshape(1, OUT_CH))
    return outp[:N]


# speedup vs baseline: 2.3862x; 1.0084x over previous
import jax
import jax.numpy as jnp
from jax import lax
from jax.experimental import pallas as pl
from jax.experimental.pallas import tpu as pltpu
from jax.experimental.pallas import tpu_sc as plsc

_f32 = jnp.float32
_i32 = jnp.int32

N = 10000
NP = 10240
E = 160000
KE = 128
NB = 80
EP = 16 * NB * KE
IN_CH, HID, OUT_CH = 256, 512, 256
RPT = NP // 16
ZR = 32
HB = NB // 2
BR = 1024
E32 = EP // 32
NV = E32 // 16

_mesh = plsc.VectorSubcoreMesh(core_axis_name="c", subcore_axis_name="s")


def _deg_body(dst_hbm, out_hbm, dstv, acc, tmp, total, shall):
    core = lax.axis_index("c")
    sub = lax.axis_index("s")
    wid = sub * 2 + core
    pltpu.sync_copy(dst_hbm.at[wid], dstv)
    zeros = jnp.zeros((16,), _f32)

    @pl.loop(0, NP // 16)
    def _(i):
        acc[pl.ds(i * 16, 16)] = zeros

    ones = jnp.ones((16,), _f32)

    @pl.loop(0, NV)
    def _(i):
        idx = dstv[pl.ds(i * 16, 16)]
        plsc.addupdate_scatter(acc, [idx], ones)

    pltpu.sync_copy(acc, shall.at[sub])
    plsc.subcore_barrier()

    @pl.loop(0, RPT // 16)
    def _(i):
        total[pl.ds(i * 16, 16)] = zeros

    for t in range(16):
        pltpu.sync_copy(shall.at[t].at[pl.ds(sub * RPT, RPT)], tmp)

        @pl.loop(0, RPT // 16)
        def _(i):
            sl = pl.ds(i * 16, 16)
            total[sl] = total[sl] + tmp[sl]

    pltpu.sync_copy(total, out_hbm.at[core].at[pl.ds(sub * RPT, RPT)])


_deg_kernel = pl.kernel(
    _deg_body,
    out_type=jax.ShapeDtypeStruct((2, NP), _f32),
    mesh=_mesh,
    compiler_params=pltpu.CompilerParams(needs_layout_passes=False),
    scratch_types=[
        pltpu.VMEM((E32,), _i32),
        pltpu.VMEM((NP,), _f32),
        pltpu.VMEM((RPT,), _f32),
        pltpu.VMEM((RPT,), _f32),
        pltpu.VMEM_SHARED((16, NP), _f32),
    ],
)


def _make_agg(n_chunks):
    cpc = n_chunks // 2

    def body(g_hbm, src_hbm, dst_hbm, out_hbm, srcv, dstv, gbuf0, gbuf1, zbuf,
             shacc, sem0, sem1):
        core = lax.axis_index("c")
        sub = lax.axis_index("s")
        my_src = src_hbm.at[sub]
        my_dst = dst_hbm.at[sub]
        zeros = jnp.zeros((16,), _f32)

        @pl.loop(0, ZR)
        def _(i):
            @pl.loop(0, 8)
            def _(k):
                zbuf[i, pl.ds(k * 16, 16)] = zeros

        def gwait(b, buf, sem):
            pltpu.make_async_copy(gc.at[srcv.at[b]], buf, sem).wait()

        def scat(b, buf):
            pltpu.sync_copy(buf, shacc.at[dstv.at[b]], add=True)

        for lc in range(cpc):
            c = core * cpc + lc
            gc = g_hbm.at[c]
            oc = out_hbm.at[c]

            pltpu.sync_copy(my_src.at[pl.ds(0, HB)], srcv)
            pltpu.sync_copy(my_dst.at[pl.ds(0, HB)], dstv)
            pltpu.async_copy(gc.at[srcv.at[0]], gbuf0, sem0)

            @pl.loop(0, RPT // ZR)
            def _(j):
                pltpu.sync_copy(zbuf, shacc.at[pl.ds(sub * RPT + j * ZR, ZR)])

            plsc.subcore_barrier()

            for h in range(2):
                if h:
                    pltpu.sync_copy(my_src.at[pl.ds(HB, HB)], srcv)
                    pltpu.sync_copy(my_dst.at[pl.ds(HB, HB)], dstv)
                    pltpu.async_copy(gc.at[srcv.at[0]], gbuf0, sem0)

                @pl.loop(0, HB // 2 - 1)
                def _(i):
                    b = i * 2
                    pltpu.async_copy(gc.at[srcv.at[b + 1]], gbuf1, sem1)
                    gwait(b, gbuf0, sem0)
                    scat(b, gbuf0)
                    pltpu.async_copy(gc.at[srcv.at[b + 2]], gbuf0, sem0)
                    gwait(b + 1, gbuf1, sem1)
                    scat(b + 1, gbuf1)

                pltpu.async_copy(gc.at[srcv.at[HB - 1]], gbuf1, sem1)
                gwait(HB - 2, gbuf0, sem0)
                scat(HB - 2, gbuf0)
                gwait(HB - 1, gbuf1, sem1)
                scat(HB - 1, gbuf1)

            plsc.subcore_barrier()

            @pl.loop(0, RPT // ZR)
            def _(j):
                off = sub * RPT + j * ZR
                pltpu.sync_copy(shacc.at[pl.ds(off, ZR)], oc.at[pl.ds(off, ZR)])

    kern = pl.kernel(
        body,
        out_type=jax.ShapeDtypeStruct((n_chunks, NP, 128), _f32),
        mesh=_mesh,
        scratch_types=[
            pltpu.VMEM((HB, KE), _i32),
            pltpu.VMEM((HB, KE), _i32),
            pltpu.VMEM((KE, 128), _f32),
            pltpu.VMEM((KE, 128), _f32),
            pltpu.VMEM((ZR, 128), _f32),
            pltpu.VMEM_SHARED((NP, 128), _f32),
            pltpu.SemaphoreType.DMA,
            pltpu.SemaphoreType.DMA,
        ],
    )
    return kern


_agg4 = _make_agg(4)
_agg2 = _make_agg(2)


def _dis_block(degp_ref, r):
    rows = lax.broadcasted_iota(_i32, (BR, 1), 0) + r * BR
    deg = (degp_ref[0, :] + degp_ref[1, :]).reshape(BR, 1)
    return jnp.where(rows < N, lax.rsqrt(deg + 1.0), 0.0)


def _m1_body(x_ref, w_ref, degp_ref, o_ref):
    dis = _dis_block(degp_ref, pl.program_id(0))
    h = jnp.dot(x_ref[...], w_ref[...], preferred_element_type=_f32)
    o_ref[0] = h * dis


_m1 = pl.pallas_call(
    _m1_body,
    grid=(NP // BR, HID // 128),
    in_specs=[
        pl.BlockSpec((BR, IN_CH), lambda r, c: (r, 0)),
        pl.BlockSpec((IN_CH, 128), lambda r, c: (0, c)),
        pl.BlockSpec((2, BR), lambda r, c: (0, r)),
    ],
    out_specs=pl.BlockSpec((1, BR, 128), lambda r, c: (c, r, 0)),
    out_shape=jax.ShapeDtypeStruct((HID // 128, NP, 128), _f32),
)


def _m2_body(p_ref, g_ref, degp_ref, b_ref, w_ref, o_ref):
    dis = _dis_block(degp_ref, pl.program_id(0))
    acc = jnp.zeros((BR, 128), _f32)
    for cc in range(HID // 128):
        t = (p_ref[cc] + g_ref[cc]) * dis + b_ref[0, cc * 128:(cc + 1) * 128][None, :]
        t = jnp.maximum(t, 0.0).astype(jnp.bfloat16)
        acc = acc + jnp.dot(t, w_ref[cc * 128:(cc + 1) * 128, :],
                            preferred_element_type=_f32)
    o_ref[0] = acc * dis


_m2 = pl.pallas_call(
    _m2_body,
    grid=(NP // BR, OUT_CH // 128),
    in_specs=[
        pl.BlockSpec((HID // 128, BR, 128), lambda r, c: (0, r, 0)),
        pl.BlockSpec((HID // 128, BR, 128), lambda r, c: (0, r, 0)),
        pl.BlockSpec((2, BR), lambda r, c: (0, r)),
        pl.BlockSpec((1, HID), lambda r, c: (0, 0)),
        pl.BlockSpec((HID, 128), lambda r, c: (0, c)),
    ],
    out_specs=pl.BlockSpec((1, BR, 128), lambda r, c: (c, r, 0)),
    out_shape=jax.ShapeDtypeStruct((OUT_CH // 128, NP, 128), _f32),
)


def _m3_body(p_ref, g_ref, degp_ref, b_ref, o_ref):
    dis = _dis_block(degp_ref, pl.program_id(0))
    t = (p_ref[0] + g_ref[0]) * dis + b_ref[0, :][None, :]
    o_ref[...] = jnp.maximum(t, 0.0)


_m3 = pl.pallas_call(
    _m3_body,
    grid=(NP // BR, OUT_CH // 128),
    in_specs=[
        pl.BlockSpec((1, BR, 128), lambda r, c: (c, r, 0)),
        pl.BlockSpec((1, BR, 128), lambda r, c: (c, r, 0)),
        pl.BlockSpec((2, BR), lambda r, c: (0, r)),
        pl.BlockSpec((1, 128), lambda r, c: (0, c)),
    ],
    out_specs=pl.BlockSpec((BR, 128), lambda r, c: (r, c)),
    out_shape=jax.ShapeDtypeStruct((NP, OUT_CH), _f32),
)


def kernel(x, edge_index, W1, b1, W2, b2):
    src = edge_index[0].astype(_i32)
    dst = edge_index[1].astype(_i32)
    pad = N + (jnp.arange(EP - E, dtype=_i32) % (NP - N))
    srcp = jnp.concatenate([src, pad]).reshape(16, NB, KE)
    dst_flat = jnp.concatenate([dst, pad])
    dstp = dst_flat.reshape(16, NB, KE)
    dst32 = dst_flat.reshape(32, E32)
    xp = jnp.concatenate([x.astype(jnp.bfloat16),
                          jnp.zeros((NP - N, IN_CH), jnp.bfloat16)])

    degp = _deg_kernel(dst32)
    g1 = _m1(xp, W1.astype(jnp.bfloat16), degp)
    p1 = _agg4(g1, srcp, dstp)
    g2 = _m2(p1, g1, degp, b1.reshape(1, HID), W2.astype(jnp.bfloat16))
    p2 = _agg2(g2, srcp, dstp)
    outp = _m3(p2, g2, degp, b2.reshape(1, OUT_CH))
    return outp[:N]
